# Initial kernel scaffold; baseline (speedup 1.0000x reference)
import jax
import jax.numpy as jnp
from jax.experimental import pallas as pl


def kernel(ndata, edge_index, edge_w, eps1, eps2, lin1_W, lin1_b, lin2_W, lin2_b, c0_W, c0_b, c1_W, c1_b, c2_W, c2_b, indices, context):
    raise NotImplementedError("write your pallas kernel here")


# speedup vs baseline: 3.6628x; 3.6628x over previous
import functools

import jax
import jax.numpy as jnp
from jax import lax
from jax.experimental import pallas as pl
from jax.experimental.pallas import tpu as pltpu
from jax.experimental.pallas import tpu_sc as plsc

N_NODES = 10000
N_EDGES = 320000
D = 128
B = 4096
IDXW = 23
IDXP = 24

NC = 2
NS = 16
NW = NC * NS
EDGES_PER_W = N_EDGES // NW
K = 80
NCHUNK = EDGES_PER_W // K
ROWS_PER_CP = 1000

SAMP_PER_W = B // NW
CH = 4
NCH = SAMP_PER_W // CH

_GDN = lax.GatherDimensionNumbers(
    offset_dims=(), collapsed_slice_dims=(0,), start_index_map=(0,))


def _bcast_lane(v, j):
  idx = jnp.full((16, 1), j, jnp.int32)
  return lax.gather(v, idx, _GDN, (1,),
                    mode=lax.GatherScatterMode.PROMISE_IN_BOUNDS)


def _make_edge_kernel(width):
  mesh = plsc.VectorSubcoreMesh(core_axis_name="c", subcore_axis_name="s")

  @functools.partial(
      pl.kernel,
      mesh=mesh,
      out_type=jax.ShapeDtypeStruct((NC * N_NODES, width), jnp.float32),
      compiler_params=pltpu.CompilerParams(use_tc_tiling_on_sc=False),
      scratch_types=[
          pltpu.VMEM((K,), jnp.int32),
          pltpu.VMEM((K,), jnp.int32),
          pltpu.VMEM((K,), jnp.float32),
          pltpu.VMEM((K, width), jnp.float32),
          pltpu.VMEM_SHARED((N_NODES, width), jnp.float32),
          pltpu.SemaphoreType.DMA,
      ],
  )
  def edge_kernel(table, src, dst, w, zeros, out, src_v, dst_v, w_v, rows_v,
                  agg_sh, sem):
    c = lax.axis_index("c")
    s = lax.axis_index("s")
    wid = c * NS + s
    @pl.when(s < N_NODES // ROWS_PER_CP)
    def _():
      pltpu.sync_copy(zeros.at[pl.ds(s * ROWS_PER_CP, ROWS_PER_CP)],
                      agg_sh.at[pl.ds(s * ROWS_PER_CP, ROWS_PER_CP)])
    plsc.subcore_barrier()
    base = wid * EDGES_PER_W

    def chunk_body(i, carry):
      off = base + i * K
      pltpu.sync_copy(src.at[pl.ds(off, K)], src_v)
      pltpu.sync_copy(dst.at[pl.ds(off, K)], dst_v)
      pltpu.sync_copy(w.at[pl.ds(off, K)], w_v)
      pltpu.async_copy(table.at[src_v], rows_v, sem).wait()

      def group_body(g, carry2):
        wg = w_v[pl.ds(g * 16, 16)]
        for j in range(16):
          wb = _bcast_lane(wg, j)
          e = g * 16 + j
          for t in range(D // 16):
            sl = rows_v[e, pl.ds(t * 16, 16)]
            rows_v[e, pl.ds(t * 16, 16)] = sl * wb
        return carry2

      lax.fori_loop(0, K // 16, group_body, 0)
      pltpu.sync_copy(rows_v, agg_sh.at[dst_v], add=True)
      return carry

    lax.fori_loop(0, NCHUNK, chunk_body, 0)
    plsc.subcore_barrier()

    @pl.when(s < N_NODES // ROWS_PER_CP)
    def _():
      pltpu.sync_copy(
          agg_sh.at[pl.ds(s * ROWS_PER_CP, ROWS_PER_CP)],
          out.at[pl.ds(c * N_NODES + s * ROWS_PER_CP, ROWS_PER_CP)])

  return edge_kernel


_edge_kernel_pad = _make_edge_kernel(D + 16)
_edge_kernel_128 = _make_edge_kernel(D)


def _make_sample_gather_kernel():
  mesh = plsc.VectorSubcoreMesh(core_axis_name="c", subcore_axis_name="s")
  nidx = SAMP_PER_W * IDXP

  @functools.partial(
      pl.kernel,
      mesh=mesh,
      out_type=jax.ShapeDtypeStruct((B, 3 * D), jnp.float32),
      compiler_params=pltpu.CompilerParams(use_tc_tiling_on_sc=False),
      scratch_types=[
          pltpu.VMEM((nidx,), jnp.int32),
          pltpu.VMEM((CH * IDXP, D), jnp.float32),
          pltpu.VMEM((SAMP_PER_W, 3 * D), jnp.float32),
          pltpu.SemaphoreType.DMA,
      ],
  )
  def gather_kernel(x, idx_flat, out, idx_v, rows_v, y_v, sem):
    c = lax.axis_index("c")
    s = lax.axis_index("s")
    wid = c * NS + s
    pltpu.sync_copy(idx_flat.at[pl.ds(wid * nidx, nidx)], idx_v)

    def chunk_body(i, carry):
      pltpu.async_copy(x.at[idx_v.at[pl.ds(i * (CH * IDXP), CH * IDXP)]],
                       rows_v, sem).wait()
      for j in range(CH):
        r0 = j * IDXP
        srow = i * CH + j
        accs = tuple(jnp.zeros((16,), jnp.float32) for _ in range(D // 16))

        def ctx_body(t, accs):
          return tuple(accs[k] + rows_v[r0 + t, pl.ds(k * 16, 16)]
                       for k in range(D // 16))

        accs = lax.fori_loop(3, IDXW, ctx_body, accs)
        for k in range(D // 16):
          sl = pl.ds(k * 16, 16)
          y_v[srow, pl.ds(k * 16, 16)] = rows_v[r0, sl]
          y_v[srow, pl.ds(D + k * 16, 16)] = rows_v[r0 + 1, sl]
          y_v[srow, pl.ds(2 * D + k * 16, 16)] = accs[k]
      return carry

    lax.fori_loop(0, NCH, chunk_body, 0)
    pltpu.sync_copy(y_v, out.at[pl.ds(wid * SAMP_PER_W, SAMP_PER_W)])

  return gather_kernel


_sample_gather_kernel = _make_sample_gather_kernel()


_BR = 1000


def _t1_body(eps_ref, x_ref, parts_ref, w_ref, b_ref, x1_ref, inv_ref):
  e = eps_ref[0, 0]
  p = parts_ref[0] + parts_ref[1]
  deg = p[:, D:D + 1]
  inv = 1.0 / jnp.where(deg == 0.0, 1.0, deg)
  h = x_ref[...] * (1.0 + e) + p[:, :D] * inv
  y = jnp.dot(h, w_ref[...], preferred_element_type=jnp.float32) + b_ref[...]
  x1_ref[...] = jnp.maximum(y, 0.0)
  inv_ref[...] = jnp.broadcast_to(inv, (_BR, D))


def _t1(eps1, ndata, parts, w1, b1):
  return pl.pallas_call(
      _t1_body,
      grid=(N_NODES // _BR,),
      in_specs=[
          pl.BlockSpec(memory_space=pltpu.SMEM),
          pl.BlockSpec((_BR, D), lambda i: (i, 0)),
          pl.BlockSpec((2, _BR, D + 16), lambda i: (0, i, 0)),
          pl.BlockSpec((D, D), lambda i: (0, 0)),
          pl.BlockSpec((1, D), lambda i: (0, 0)),
      ],
      out_specs=[
          pl.BlockSpec((_BR, D), lambda i: (i, 0)),
          pl.BlockSpec((_BR, D), lambda i: (i, 0)),
      ],
      out_shape=[
          jax.ShapeDtypeStruct((N_NODES, D), jnp.float32),
          jax.ShapeDtypeStruct((N_NODES, D), jnp.float32),
      ],
  )(eps1, ndata, parts, w1, b1)


def _t2_body(eps_ref, x_ref, parts_ref, inv_ref, w_ref, b_ref, x2_ref):
  e = eps_ref[0, 0]
  p = parts_ref[0] + parts_ref[1]
  h = x_ref[...] * (1.0 + e) + p * inv_ref[...]
  y = jnp.dot(h, w_ref[...], preferred_element_type=jnp.float32) + b_ref[...]

  @pl.when(pl.program_id(0) == 0)
  def _():
    y0 = jnp.where(
        lax.broadcasted_iota(jnp.int32, (_BR, D), 0) == 0, 0.0, y)
    x2_ref[...] = y0

  @pl.when(pl.program_id(0) != 0)
  def _():
    x2_ref[...] = y


def _t2(eps2, x1, parts2, inv, w2, b2):
  return pl.pallas_call(
      _t2_body,
      grid=(N_NODES // _BR,),
      in_specs=[
          pl.BlockSpec(memory_space=pltpu.SMEM),
          pl.BlockSpec((_BR, D), lambda i: (i, 0)),
          pl.BlockSpec((2, _BR, D), lambda i: (0, i, 0)),
          pl.BlockSpec((_BR, D), lambda i: (i, 0)),
          pl.BlockSpec((D, D), lambda i: (0, 0)),
          pl.BlockSpec((1, D), lambda i: (0, 0)),
      ],
      out_specs=pl.BlockSpec((_BR, D), lambda i: (i, 0)),
      out_shape=jax.ShapeDtypeStruct((N_NODES, D), jnp.float32),
  )(eps2, x1, parts2, inv, w2, b2)


def _t3_body(flag_ref, idx_ref, y_ref, c0w_ref, c0b_ref, c1w_ref, c1b_ref,
             c2w_ref, c2b_ref, out_ref):
  flag = flag_ref[0, 0]
  mask = (idx_ref[...][:, 3:] > 0).astype(jnp.float32)
  norm = jnp.sum(mask, axis=1, keepdims=True)
  inv = 1.0 / jnp.where(norm == 0.0, 1.0, norm)
  y = y_ref[...]
  ctx = y[:, 2 * D:] * (inv * flag)
  yy = jnp.concatenate([y[:, :2 * D], ctx], axis=1)
  h = jnp.maximum(
      jnp.dot(yy, c0w_ref[...], preferred_element_type=jnp.float32)
      + c0b_ref[...], 0.0)
  h = jnp.maximum(
      jnp.dot(h, c1w_ref[...], preferred_element_type=jnp.float32)
      + c1b_ref[...], 0.0)
  out_ref[...] = (
      jnp.dot(h, c2w_ref[...], preferred_element_type=jnp.float32)
      + c2b_ref[...])


def _t3(flag, indices, y, c0w, c0b, c1w, c1b, c2w, c2b):
  return pl.pallas_call(
      _t3_body,
      in_specs=[
          pl.BlockSpec(memory_space=pltpu.SMEM),
          pl.BlockSpec((B, IDXW), lambda: (0, 0)),
          pl.BlockSpec((B, 3 * D), lambda: (0, 0)),
          pl.BlockSpec((3 * D, D), lambda: (0, 0)),
          pl.BlockSpec((1, D), lambda: (0, 0)),
          pl.BlockSpec((D, D // 2), lambda: (0, 0)),
          pl.BlockSpec((1, D // 2), lambda: (0, 0)),
          pl.BlockSpec((D // 2, 1), lambda: (0, 0)),
          pl.BlockSpec((1, 1), lambda: (0, 0)),
      ],
      out_specs=pl.BlockSpec((B, 1), lambda: (0, 0)),
      out_shape=jax.ShapeDtypeStruct((B, 1), jnp.float32),
  )(flag, indices, y, c0w, c0b, c1w, c1b, c2w, c2b)


def kernel(ndata, edge_index, edge_w, eps1, eps2, lin1_W, lin1_b, lin2_W,
           lin2_b, c0_W, c0_b, c1_W, c1_b, c2_W, c2_b, indices, context):
  src = edge_index[0]
  dst = edge_index[1]
  xpad = jnp.concatenate(
      [ndata, jnp.ones((N_NODES, 1), jnp.float32),
       jnp.zeros((N_NODES, 15), jnp.float32)], axis=1)
  zeros_pad = jnp.zeros((N_NODES, D + 16), jnp.float32)
  zeros_128 = jnp.zeros((N_NODES, D), jnp.float32)

  parts1 = _edge_kernel_pad(xpad, src, dst, edge_w, zeros_pad)
  parts1 = parts1.reshape(NC, N_NODES, D + 16)

  eps1_2d = jnp.reshape(eps1, (1, 1))
  eps2_2d = jnp.reshape(eps2, (1, 1))
  x1, inv = _t1(eps1_2d, ndata, parts1, lin1_W, jnp.reshape(lin1_b, (1, D)))

  parts2 = _edge_kernel_128(x1, src, dst, edge_w, zeros_128)
  parts2 = parts2.reshape(NC, N_NODES, D)
  x2 = _t2(eps2_2d, x1, parts2, inv, lin2_W, jnp.reshape(lin2_b, (1, D)))

  idx_pad = jnp.concatenate(
      [indices, jnp.zeros((B, IDXP - IDXW), jnp.int32)], axis=1)
  y = _sample_gather_kernel(x2, idx_pad.reshape(-1))

  flag = jnp.reshape(jnp.asarray(context, jnp.float32), (1, 1))
  out = _t3(flag, indices, y, c0_W, jnp.reshape(c0_b, (1, D)), c1_W,
            jnp.reshape(c1_b, (1, D // 2)), c2_W, jnp.reshape(c2_b, (1, 1)))
  return out


# speedup vs baseline: 6.9589x; 1.8999x over previous
import functools

import jax
import jax.numpy as jnp
from jax import lax
from jax.experimental import pallas as pl
from jax.experimental.pallas import tpu as pltpu
from jax.experimental.pallas import tpu_sc as plsc

N_NODES = 10000
N_EDGES = 320000
D = 128
B = 4096
IDXW = 23
IDXP = 24

NC = 2
NS = 16
NW = NC * NS
EDGES_PER_W = N_EDGES // NW
K = 80
NCHUNK = EDGES_PER_W // K
ROWS_PER_CP = 1000

SAMP_PER_W = B // NW
CH = 4
NCH = SAMP_PER_W // CH

_GDN = lax.GatherDimensionNumbers(
    offset_dims=(), collapsed_slice_dims=(0,), start_index_map=(0,))


def _bcast_lane(v, j):
  idx = jnp.full((16, 1), j, jnp.int32)
  return lax.gather(v, idx, _GDN, (1,),
                    mode=lax.GatherScatterMode.PROMISE_IN_BOUNDS)


def _make_edge_kernel(width):
  mesh = plsc.VectorSubcoreMesh(core_axis_name="c", subcore_axis_name="s")

  @functools.partial(
      pl.kernel,
      mesh=mesh,
      out_type=jax.ShapeDtypeStruct((NC * N_NODES, width), jnp.float32),
      compiler_params=pltpu.CompilerParams(use_tc_tiling_on_sc=False),
      scratch_types=[
          pltpu.VMEM((NCHUNK, K), jnp.int32),
          pltpu.VMEM((K,), jnp.int32),
          pltpu.VMEM((K,), jnp.int32),
          pltpu.VMEM((K,), jnp.float32),
          pltpu.VMEM((K,), jnp.float32),
          pltpu.VMEM((K, width), jnp.float32),
          pltpu.VMEM((K, width), jnp.float32),
          pltpu.VMEM_SHARED((N_NODES, width), jnp.float32),
          pltpu.SemaphoreType.DMA,
          pltpu.SemaphoreType.DMA,
          pltpu.SemaphoreType.DMA,
          pltpu.SemaphoreType.DMA,
          pltpu.SemaphoreType.DMA,
          pltpu.SemaphoreType.DMA,
      ],
  )
  def edge_kernel(table, src, dst3, w, zeros, out, dst_all, src0, src1, w0,
                  w1, rows0, rows1, agg_sh, gsem0, gsem1, ssem0, ssem1, wsem0,
                  wsem1):
    c = lax.axis_index("c")
    s = lax.axis_index("s")
    wid = c * NS + s
    @pl.when(s < N_NODES // ROWS_PER_CP)
    def _():
      pltpu.sync_copy(zeros.at[pl.ds(s * ROWS_PER_CP, ROWS_PER_CP)],
                      agg_sh.at[pl.ds(s * ROWS_PER_CP, ROWS_PER_CP)])
    base = wid * EDGES_PER_W
    pltpu.sync_copy(dst3.at[wid], dst_all)
    plsc.subcore_barrier()

    def src_start(i, buf, sem):
      pltpu.async_copy(src.at[pl.ds(base + i * K, K)], buf, sem)

    def src_wait(i, buf, sem):
      pltpu.make_async_copy(src.at[pl.ds(base + i * K, K)], buf, sem).wait()

    def w_start(i, buf, sem):
      pltpu.async_copy(w.at[pl.ds(base + i * K, K)], buf, sem)

    def w_wait(i, buf, sem):
      pltpu.make_async_copy(w.at[pl.ds(base + i * K, K)], buf, sem).wait()

    def g_start(sbuf, buf, sem):
      pltpu.async_copy(table.at[sbuf], buf, sem)

    def g_wait(sbuf, buf, sem):
      pltpu.make_async_copy(table.at[sbuf], buf, sem).wait()

    def mul_scatter(i, buf, wbuf):
      def group_body(g, carry2):
        wg = wbuf[pl.ds(g * 16, 16)]
        for j in range(16):
          wb = _bcast_lane(wg, j)
          e = g * 16 + j
          for t in range(D // 16):
            sl = buf[e, pl.ds(t * 16, 16)]
            buf[e, pl.ds(t * 16, 16)] = sl * wb
        return carry2

      lax.fori_loop(0, K // 16, group_body, 0)
      pltpu.sync_copy(buf, agg_sh.at[dst_all.at[i]], add=True)

    src_start(0, src0, ssem0)
    w_start(0, w0, wsem0)
    src_wait(0, src0, ssem0)
    g_start(src0, rows0, gsem0)
    src_start(1, src1, ssem1)
    w_start(1, w1, wsem1)

    def step(i, sbufs, wbufs, rbufs, ssems, wsems, gsems):
      src_wait(i + 1, sbufs[1], ssems[1])
      g_start(sbufs[1], rbufs[1], gsems[1])
      g_wait(sbufs[0], rbufs[0], gsems[0])

      @pl.when(i + 2 < NCHUNK)
      def _():
        src_start(i + 2, sbufs[0], ssems[0])

      w_wait(i, wbufs[0], wsems[0])
      mul_scatter(i, rbufs[0], wbufs[0])

      @pl.when(i + 2 < NCHUNK)
      def _():
        w_start(i + 2, wbufs[0], wsems[0])

    def pair_body(i2, carry):
      i = i2 * 2
      step(i, (src0, src1), (w0, w1), (rows0, rows1), (ssem0, ssem1),
           (wsem0, wsem1), (gsem0, gsem1))
      step(i + 1, (src1, src0), (w1, w0), (rows1, rows0), (ssem1, ssem0),
           (wsem1, wsem0), (gsem1, gsem0))
      return carry

    lax.fori_loop(0, (NCHUNK - 1) // 2, pair_body, 0)
    g_wait(src0, rows0, gsem0)
    w_wait(NCHUNK - 1, w0, wsem0)
    mul_scatter(NCHUNK - 1, rows0, w0)
    plsc.subcore_barrier()

    @pl.when(s < N_NODES // ROWS_PER_CP)
    def _():
      pltpu.sync_copy(
          agg_sh.at[pl.ds(s * ROWS_PER_CP, ROWS_PER_CP)],
          out.at[pl.ds(c * N_NODES + s * ROWS_PER_CP, ROWS_PER_CP)])

  return edge_kernel


_edge_kernel_pad = _make_edge_kernel(D + 16)
_edge_kernel_128 = _make_edge_kernel(D)


def _make_sample_gather_kernel():
  mesh = plsc.VectorSubcoreMesh(core_axis_name="c", subcore_axis_name="s")
  nidx = SAMP_PER_W * IDXP
  chi = CH * IDXP

  @functools.partial(
      pl.kernel,
      mesh=mesh,
      out_type=jax.ShapeDtypeStruct((B, 3 * D), jnp.float32),
      compiler_params=pltpu.CompilerParams(use_tc_tiling_on_sc=False),
      scratch_types=[
          pltpu.VMEM((nidx,), jnp.int32),
          pltpu.VMEM((chi, D), jnp.float32),
          pltpu.VMEM((chi, D), jnp.float32),
          pltpu.VMEM((SAMP_PER_W, 3 * D), jnp.float32),
          pltpu.SemaphoreType.DMA,
          pltpu.SemaphoreType.DMA,
      ],
  )
  def gather_kernel(x, idx_flat, out, idx_v, rows0, rows1, y_v, sem0, sem1):
    c = lax.axis_index("c")
    s = lax.axis_index("s")
    wid = c * NS + s
    pltpu.sync_copy(idx_flat.at[pl.ds(wid * nidx, nidx)], idx_v)

    def g_start(i, buf, sem):
      pltpu.async_copy(x.at[idx_v.at[pl.ds(i * chi, chi)]], buf, sem)

    def g_wait(i, buf, sem):
      pltpu.make_async_copy(x.at[idx_v.at[pl.ds(i * chi, chi)]], buf,
                            sem).wait()

    def process(i, buf):
      for j in range(CH):
        r0 = j * IDXP
        srow = i * CH + j
        for k in range(D // 16):
          sl = pl.ds(k * 16, 16)
          acc = buf[r0 + 3, sl]
          for t in range(4, IDXW):
            acc = acc + buf[r0 + t, sl]
          y_v[srow, pl.ds(k * 16, 16)] = buf[r0, sl]
          y_v[srow, pl.ds(D + k * 16, 16)] = buf[r0 + 1, sl]
          y_v[srow, pl.ds(2 * D + k * 16, 16)] = acc

    g_start(0, rows0, sem0)

    def pair_body(i2, carry):
      i = i2 * 2
      g_start(i + 1, rows1, sem1)
      g_wait(i, rows0, sem0)
      process(i, rows0)

      @pl.when(i + 2 < NCH)
      def _():
        g_start(i + 2, rows0, sem0)

      g_wait(i + 1, rows1, sem1)
      process(i + 1, rows1)
      return carry

    lax.fori_loop(0, NCH // 2, pair_body, 0)
    pltpu.sync_copy(y_v, out.at[pl.ds(wid * SAMP_PER_W, SAMP_PER_W)])

  return gather_kernel


_sample_gather_kernel = _make_sample_gather_kernel()


_BR = 1000


def _t1_body(eps_ref, x_ref, parts_ref, w_ref, b_ref, x1_ref, inv_ref):
  e = eps_ref[0, 0]
  p = parts_ref[0] + parts_ref[1]
  deg = p[:, D:D + 1]
  inv = 1.0 / jnp.where(deg == 0.0, 1.0, deg)
  h = x_ref[...] * (1.0 + e) + p[:, :D] * inv
  y = jnp.dot(h, w_ref[...], preferred_element_type=jnp.float32) + b_ref[...]
  x1_ref[...] = jnp.maximum(y, 0.0)
  inv_ref[...] = jnp.broadcast_to(inv, (_BR, D))


def _t1(eps1, ndata, parts, w1, b1):
  return pl.pallas_call(
      _t1_body,
      grid=(N_NODES // _BR,),
      in_specs=[
          pl.BlockSpec(memory_space=pltpu.SMEM),
          pl.BlockSpec((_BR, D), lambda i: (i, 0)),
          pl.BlockSpec((2, _BR, D + 16), lambda i: (0, i, 0)),
          pl.BlockSpec((D, D), lambda i: (0, 0)),
          pl.BlockSpec((1, D), lambda i: (0, 0)),
      ],
      out_specs=[
          pl.BlockSpec((_BR, D), lambda i: (i, 0)),
          pl.BlockSpec((_BR, D), lambda i: (i, 0)),
      ],
      out_shape=[
          jax.ShapeDtypeStruct((N_NODES, D), jnp.float32),
          jax.ShapeDtypeStruct((N_NODES, D), jnp.float32),
      ],
  )(eps1, ndata, parts, w1, b1)


def _t2_body(eps_ref, x_ref, parts_ref, inv_ref, w_ref, b_ref, x2_ref):
  e = eps_ref[0, 0]
  p = parts_ref[0] + parts_ref[1]
  h = x_ref[...] * (1.0 + e) + p * inv_ref[...]
  y = jnp.dot(h, w_ref[...], preferred_element_type=jnp.float32) + b_ref[...]

  @pl.when(pl.program_id(0) == 0)
  def _():
    y0 = jnp.where(
        lax.broadcasted_iota(jnp.int32, (_BR, D), 0) == 0, 0.0, y)
    x2_ref[...] = y0

  @pl.when(pl.program_id(0) != 0)
  def _():
    x2_ref[...] = y


def _t2(eps2, x1, parts2, inv, w2, b2):
  return pl.pallas_call(
      _t2_body,
      grid=(N_NODES // _BR,),
      in_specs=[
          pl.BlockSpec(memory_space=pltpu.SMEM),
          pl.BlockSpec((_BR, D), lambda i: (i, 0)),
          pl.BlockSpec((2, _BR, D), lambda i: (0, i, 0)),
          pl.BlockSpec((_BR, D), lambda i: (i, 0)),
          pl.BlockSpec((D, D), lambda i: (0, 0)),
          pl.BlockSpec((1, D), lambda i: (0, 0)),
      ],
      out_specs=pl.BlockSpec((_BR, D), lambda i: (i, 0)),
      out_shape=jax.ShapeDtypeStruct((N_NODES, D), jnp.float32),
  )(eps2, x1, parts2, inv, w2, b2)


def _t3_body(flag_ref, idx_ref, y_ref, c0w_ref, c0b_ref, c1w_ref, c1b_ref,
             c2w_ref, c2b_ref, out_ref):
  flag = flag_ref[0, 0]
  mask = (idx_ref[...][:, 3:] > 0).astype(jnp.float32)
  norm = jnp.sum(mask, axis=1, keepdims=True)
  inv = 1.0 / jnp.where(norm == 0.0, 1.0, norm)
  y = y_ref[...]
  ctx = y[:, 2 * D:] * (inv * flag)
  yy = jnp.concatenate([y[:, :2 * D], ctx], axis=1)
  h = jnp.maximum(
      jnp.dot(yy, c0w_ref[...], preferred_element_type=jnp.float32)
      + c0b_ref[...], 0.0)
  h = jnp.maximum(
      jnp.dot(h, c1w_ref[...], preferred_element_type=jnp.float32)
      + c1b_ref[...], 0.0)
  out_ref[...] = (
      jnp.dot(h, c2w_ref[...], preferred_element_type=jnp.float32)
      + c2b_ref[...])


def _t3(flag, indices, y, c0w, c0b, c1w, c1b, c2w, c2b):
  return pl.pallas_call(
      _t3_body,
      in_specs=[
          pl.BlockSpec(memory_space=pltpu.SMEM),
          pl.BlockSpec((B, IDXW), lambda: (0, 0)),
          pl.BlockSpec((B, 3 * D), lambda: (0, 0)),
          pl.BlockSpec((3 * D, D), lambda: (0, 0)),
          pl.BlockSpec((1, D), lambda: (0, 0)),
          pl.BlockSpec((D, D // 2), lambda: (0, 0)),
          pl.BlockSpec((1, D // 2), lambda: (0, 0)),
          pl.BlockSpec((D // 2, 1), lambda: (0, 0)),
          pl.BlockSpec((1, 1), lambda: (0, 0)),
      ],
      out_specs=pl.BlockSpec((B, 1), lambda: (0, 0)),
      out_shape=jax.ShapeDtypeStruct((B, 1), jnp.float32),
  )(flag, indices, y, c0w, c0b, c1w, c1b, c2w, c2b)


def kernel(ndata, edge_index, edge_w, eps1, eps2, lin1_W, lin1_b, lin2_W,
           lin2_b, c0_W, c0_b, c1_W, c1_b, c2_W, c2_b, indices, context):
  src = edge_index[0]
  dst3 = edge_index[1].reshape(NW, NCHUNK, K)
  xpad = jnp.concatenate(
      [ndata, jnp.ones((N_NODES, 1), jnp.float32),
       jnp.zeros((N_NODES, 15), jnp.float32)], axis=1)
  zeros_pad = jnp.zeros((N_NODES, D + 16), jnp.float32)
  zeros_128 = jnp.zeros((N_NODES, D), jnp.float32)

  parts1 = _edge_kernel_pad(xpad, src, dst3, edge_w, zeros_pad)
  parts1 = parts1.reshape(NC, N_NODES, D + 16)

  eps1_2d = jnp.reshape(eps1, (1, 1))
  eps2_2d = jnp.reshape(eps2, (1, 1))
  x1, inv = _t1(eps1_2d, ndata, parts1, lin1_W, jnp.reshape(lin1_b, (1, D)))

  parts2 = _edge_kernel_128(x1, src, dst3, edge_w, zeros_128)
  parts2 = parts2.reshape(NC, N_NODES, D)
  x2 = _t2(eps2_2d, x1, parts2, inv, lin2_W, jnp.reshape(lin2_b, (1, D)))

  idx_pad = jnp.concatenate(
      [indices, jnp.zeros((B, IDXP - IDXW), jnp.int32)], axis=1)
  y = _sample_gather_kernel(x2, idx_pad.reshape(-1))

  flag = jnp.reshape(jnp.asarray(context, jnp.float32), (1, 1))
  out = _t3(flag, indices, y, c0_W, jnp.reshape(c0_b, (1, D)), c1_W,
            jnp.reshape(c1_b, (1, D // 2)), c2_W, jnp.reshape(c2_b, (1, 1)))
  return out


# speedup vs baseline: 6.9785x; 1.0028x over previous
import functools

import jax
import jax.numpy as jnp
from jax import lax
from jax.experimental import pallas as pl
from jax.experimental.pallas import tpu as pltpu
from jax.experimental.pallas import tpu_sc as plsc

N_NODES = 10000
N_EDGES = 320000
D = 128
B = 4096
IDXW = 23
IDXP = 24

NC = 2
NS = 16
NW = NC * NS
EDGES_PER_W = N_EDGES // NW
K = 80
NCHUNK = EDGES_PER_W // K
ROWS_PER_CP = 1000

SAMP_PER_W = B // NW
CH = 4
NCH = SAMP_PER_W // CH

_GDN = lax.GatherDimensionNumbers(
    offset_dims=(), collapsed_slice_dims=(0,), start_index_map=(0,))


def _bcast_lane(v, j):
  idx = jnp.full((16, 1), j, jnp.int32)
  return lax.gather(v, idx, _GDN, (1,),
                    mode=lax.GatherScatterMode.PROMISE_IN_BOUNDS)


def _make_edge_kernel(width):
  mesh = plsc.VectorSubcoreMesh(core_axis_name="c", subcore_axis_name="s")

  @functools.partial(
      pl.kernel,
      mesh=mesh,
      out_type=jax.ShapeDtypeStruct((NC * N_NODES, width), jnp.float32),
      compiler_params=pltpu.CompilerParams(use_tc_tiling_on_sc=False),
      scratch_types=[
          pltpu.VMEM((NCHUNK, K), jnp.int32),
          pltpu.VMEM((K,), jnp.int32),
          pltpu.VMEM((K,), jnp.int32),
          pltpu.VMEM((K,), jnp.float32),
          pltpu.VMEM((K,), jnp.float32),
          pltpu.VMEM((K, width), jnp.float32),
          pltpu.VMEM((K, width), jnp.float32),
          pltpu.VMEM_SHARED((N_NODES, width), jnp.float32),
          pltpu.SemaphoreType.DMA,
          pltpu.SemaphoreType.DMA,
          pltpu.SemaphoreType.DMA,
          pltpu.SemaphoreType.DMA,
          pltpu.SemaphoreType.DMA,
          pltpu.SemaphoreType.DMA,
      ],
  )
  def edge_kernel(table, src, dst3, w, zeros, out, dst_all, src0, src1, w0,
                  w1, rows0, rows1, agg_sh, gsem0, gsem1, ssem0, ssem1, wsem0,
                  wsem1):
    c = lax.axis_index("c")
    s = lax.axis_index("s")
    wid = c * NS + s
    @pl.when(s < N_NODES // ROWS_PER_CP)
    def _():
      pltpu.sync_copy(zeros.at[pl.ds(s * ROWS_PER_CP, ROWS_PER_CP)],
                      agg_sh.at[pl.ds(s * ROWS_PER_CP, ROWS_PER_CP)])
    base = wid * EDGES_PER_W
    pltpu.sync_copy(dst3.at[wid], dst_all)
    plsc.subcore_barrier()

    def src_start(i, buf, sem):
      pltpu.async_copy(src.at[pl.ds(base + i * K, K)], buf, sem)

    def src_wait(i, buf, sem):
      pltpu.make_async_copy(src.at[pl.ds(base + i * K, K)], buf, sem).wait()

    def w_start(i, buf, sem):
      pltpu.async_copy(w.at[pl.ds(base + i * K, K)], buf, sem)

    def w_wait(i, buf, sem):
      pltpu.make_async_copy(w.at[pl.ds(base + i * K, K)], buf, sem).wait()

    def g_start(sbuf, buf, sem):
      pltpu.async_copy(table.at[sbuf], buf, sem)

    def g_wait(sbuf, buf, sem):
      pltpu.make_async_copy(table.at[sbuf], buf, sem).wait()

    def mul_scatter(i, buf, wbuf):
      def group_body(g, carry2):
        wg = wbuf[pl.ds(g * 16, 16)]
        for j in range(16):
          wb = _bcast_lane(wg, j)
          e = g * 16 + j
          for t in range(D // 16):
            sl = buf[e, pl.ds(t * 16, 16)]
            buf[e, pl.ds(t * 16, 16)] = sl * wb
        return carry2

      lax.fori_loop(0, K // 16, group_body, 0)
      pltpu.sync_copy(buf, agg_sh.at[dst_all.at[i]], add=True)

    src_start(0, src0, ssem0)
    w_start(0, w0, wsem0)
    src_wait(0, src0, ssem0)
    g_start(src0, rows0, gsem0)
    src_start(1, src1, ssem1)
    w_start(1, w1, wsem1)

    def step(i, sbufs, wbufs, rbufs, ssems, wsems, gsems):
      src_wait(i + 1, sbufs[1], ssems[1])
      g_start(sbufs[1], rbufs[1], gsems[1])
      g_wait(sbufs[0], rbufs[0], gsems[0])

      @pl.when(i + 2 < NCHUNK)
      def _():
        src_start(i + 2, sbufs[0], ssems[0])

      w_wait(i, wbufs[0], wsems[0])
      mul_scatter(i, rbufs[0], wbufs[0])

      @pl.when(i + 2 < NCHUNK)
      def _():
        w_start(i + 2, wbufs[0], wsems[0])

    def pair_body(i2, carry):
      i = i2 * 2
      step(i, (src0, src1), (w0, w1), (rows0, rows1), (ssem0, ssem1),
           (wsem0, wsem1), (gsem0, gsem1))
      step(i + 1, (src1, src0), (w1, w0), (rows1, rows0), (ssem1, ssem0),
           (wsem1, wsem0), (gsem1, gsem0))
      return carry

    lax.fori_loop(0, (NCHUNK - 1) // 2, pair_body, 0)
    g_wait(src0, rows0, gsem0)
    w_wait(NCHUNK - 1, w0, wsem0)
    mul_scatter(NCHUNK - 1, rows0, w0)
    plsc.subcore_barrier()

    @pl.when(s < N_NODES // ROWS_PER_CP)
    def _():
      pltpu.sync_copy(
          agg_sh.at[pl.ds(s * ROWS_PER_CP, ROWS_PER_CP)],
          out.at[pl.ds(c * N_NODES + s * ROWS_PER_CP, ROWS_PER_CP)])

  return edge_kernel


_edge_kernel_pad = _make_edge_kernel(D + 16)
_edge_kernel_128 = _make_edge_kernel(D)


def _make_sample_gather_kernel():
  mesh = plsc.VectorSubcoreMesh(core_axis_name="c", subcore_axis_name="s")
  nidx = SAMP_PER_W * IDXP
  chi = CH * IDXP
  nring = 4

  @functools.partial(
      pl.kernel,
      mesh=mesh,
      out_type=jax.ShapeDtypeStruct((B, 3 * D), jnp.float32),
      compiler_params=pltpu.CompilerParams(use_tc_tiling_on_sc=False),
      scratch_types=[
          pltpu.VMEM((nidx,), jnp.int32),
          [pltpu.VMEM((chi, D), jnp.float32) for _ in range(nring)],
          pltpu.VMEM((SAMP_PER_W, 3 * D), jnp.float32),
          [pltpu.SemaphoreType.DMA for _ in range(nring)],
      ],
  )
  def gather_kernel(x, idx_flat, out, idx_v, rows, y_v, sems):
    c = lax.axis_index("c")
    s = lax.axis_index("s")
    wid = c * NS + s
    pltpu.sync_copy(idx_flat.at[pl.ds(wid * nidx, nidx)], idx_v)

    def g_start(i, buf, sem):
      pltpu.async_copy(x.at[idx_v.at[pl.ds(i * chi, chi)]], buf, sem)

    def g_wait(i, buf, sem):
      pltpu.make_async_copy(x.at[idx_v.at[pl.ds(i * chi, chi)]], buf,
                            sem).wait()

    def process(i, buf):
      for j in range(CH):
        r0 = j * IDXP
        srow = i * CH + j
        accs = tuple(jnp.zeros((16,), jnp.float32) for _ in range(D // 16))

        def ctx_body(t, accs):
          return tuple(accs[k] + buf[r0 + t, pl.ds(k * 16, 16)]
                       for k in range(D // 16))

        accs = lax.fori_loop(3, IDXW, ctx_body, accs)
        for k in range(D // 16):
          sl = pl.ds(k * 16, 16)
          y_v[srow, pl.ds(k * 16, 16)] = buf[r0, sl]
          y_v[srow, pl.ds(D + k * 16, 16)] = buf[r0 + 1, sl]
          y_v[srow, pl.ds(2 * D + k * 16, 16)] = accs[k]

    for b in range(nring):
      g_start(b, rows[b], sems[b])

    def ring_body(i4, carry):
      for b in range(nring):
        i = i4 * nring + b
        g_wait(i, rows[b], sems[b])
        process(i, rows[b])

        @pl.when(i + nring < NCH)
        def _():
          g_start(i + nring, rows[b], sems[b])
      return carry

    lax.fori_loop(0, NCH // nring, ring_body, 0)
    pltpu.sync_copy(y_v, out.at[pl.ds(wid * SAMP_PER_W, SAMP_PER_W)])

  return gather_kernel


_sample_gather_kernel = _make_sample_gather_kernel()


_BR = 1000


def _t1_body(eps_ref, x_ref, parts_ref, w_ref, b_ref, x1_ref, inv_ref):
  e = eps_ref[0, 0]
  p = parts_ref[0] + parts_ref[1]
  deg = p[:, D:D + 1]
  inv = 1.0 / jnp.where(deg == 0.0, 1.0, deg)
  h = x_ref[...] * (1.0 + e) + p[:, :D] * inv
  y = jnp.dot(h, w_ref[...], preferred_element_type=jnp.float32) + b_ref[...]
  x1_ref[...] = jnp.maximum(y, 0.0)
  inv_ref[...] = jnp.broadcast_to(inv, (_BR, D))


def _t1(eps1, ndata, parts, w1, b1):
  return pl.pallas_call(
      _t1_body,
      grid=(N_NODES // _BR,),
      in_specs=[
          pl.BlockSpec(memory_space=pltpu.SMEM),
          pl.BlockSpec((_BR, D), lambda i: (i, 0)),
          pl.BlockSpec((2, _BR, D + 16), lambda i: (0, i, 0)),
          pl.BlockSpec((D, D), lambda i: (0, 0)),
          pl.BlockSpec((1, D), lambda i: (0, 0)),
      ],
      out_specs=[
          pl.BlockSpec((_BR, D), lambda i: (i, 0)),
          pl.BlockSpec((_BR, D), lambda i: (i, 0)),
      ],
      out_shape=[
          jax.ShapeDtypeStruct((N_NODES, D), jnp.float32),
          jax.ShapeDtypeStruct((N_NODES, D), jnp.float32),
      ],
  )(eps1, ndata, parts, w1, b1)


def _t2_body(eps_ref, x_ref, parts_ref, inv_ref, w_ref, b_ref, x2_ref):
  e = eps_ref[0, 0]
  p = parts_ref[0] + parts_ref[1]
  h = x_ref[...] * (1.0 + e) + p * inv_ref[...]
  y = jnp.dot(h, w_ref[...], preferred_element_type=jnp.float32) + b_ref[...]

  @pl.when(pl.program_id(0) == 0)
  def _():
    y0 = jnp.where(
        lax.broadcasted_iota(jnp.int32, (_BR, D), 0) == 0, 0.0, y)
    x2_ref[...] = y0

  @pl.when(pl.program_id(0) != 0)
  def _():
    x2_ref[...] = y


def _t2(eps2, x1, parts2, inv, w2, b2):
  return pl.pallas_call(
      _t2_body,
      grid=(N_NODES // _BR,),
      in_specs=[
          pl.BlockSpec(memory_space=pltpu.SMEM),
          pl.BlockSpec((_BR, D), lambda i: (i, 0)),
          pl.BlockSpec((2, _BR, D), lambda i: (0, i, 0)),
          pl.BlockSpec((_BR, D), lambda i: (i, 0)),
          pl.BlockSpec((D, D), lambda i: (0, 0)),
          pl.BlockSpec((1, D), lambda i: (0, 0)),
      ],
      out_specs=pl.BlockSpec((_BR, D), lambda i: (i, 0)),
      out_shape=jax.ShapeDtypeStruct((N_NODES, D), jnp.float32),
  )(eps2, x1, parts2, inv, w2, b2)


def _t3_body(flag_ref, idx_ref, y_ref, c0w_ref, c0b_ref, c1w_ref, c1b_ref,
             c2w_ref, c2b_ref, out_ref):
  flag = flag_ref[0, 0]
  mask = (idx_ref[...][:, 3:] > 0).astype(jnp.float32)
  norm = jnp.sum(mask, axis=1, keepdims=True)
  inv = 1.0 / jnp.where(norm == 0.0, 1.0, norm)
  y = y_ref[...]
  ctx = y[:, 2 * D:] * (inv * flag)
  yy = jnp.concatenate([y[:, :2 * D], ctx], axis=1)
  h = jnp.maximum(
      jnp.dot(yy, c0w_ref[...], preferred_element_type=jnp.float32)
      + c0b_ref[...], 0.0)
  h = jnp.maximum(
      jnp.dot(h, c1w_ref[...], preferred_element_type=jnp.float32)
      + c1b_ref[...], 0.0)
  out_ref[...] = (
      jnp.dot(h, c2w_ref[...], preferred_element_type=jnp.float32)
      + c2b_ref[...])


def _t3(flag, indices, y, c0w, c0b, c1w, c1b, c2w, c2b):
  return pl.pallas_call(
      _t3_body,
      in_specs=[
          pl.BlockSpec(memory_space=pltpu.SMEM),
          pl.BlockSpec((B, IDXW), lambda: (0, 0)),
          pl.BlockSpec((B, 3 * D), lambda: (0, 0)),
          pl.BlockSpec((3 * D, D), lambda: (0, 0)),
          pl.BlockSpec((1, D), lambda: (0, 0)),
          pl.BlockSpec((D, D // 2), lambda: (0, 0)),
          pl.BlockSpec((1, D // 2), lambda: (0, 0)),
          pl.BlockSpec((D // 2, 1), lambda: (0, 0)),
          pl.BlockSpec((1, 1), lambda: (0, 0)),
      ],
      out_specs=pl.BlockSpec((B, 1), lambda: (0, 0)),
      out_shape=jax.ShapeDtypeStruct((B, 1), jnp.float32),
  )(flag, indices, y, c0w, c0b, c1w, c1b, c2w, c2b)


def kernel(ndata, edge_index, edge_w, eps1, eps2, lin1_W, lin1_b, lin2_W,
           lin2_b, c0_W, c0_b, c1_W, c1_b, c2_W, c2_b, indices, context):
  src = edge_index[0]
  dst3 = edge_index[1].reshape(NW, NCHUNK, K)
  xpad = jnp.concatenate(
      [ndata, jnp.ones((N_NODES, 1), jnp.float32),
       jnp.zeros((N_NODES, 15), jnp.float32)], axis=1)
  zeros_pad = jnp.zeros((N_NODES, D + 16), jnp.float32)
  zeros_128 = jnp.zeros((N_NODES, D), jnp.float32)

  parts1 = _edge_kernel_pad(xpad, src, dst3, edge_w, zeros_pad)
  parts1 = parts1.reshape(NC, N_NODES, D + 16)

  eps1_2d = jnp.reshape(eps1, (1, 1))
  eps2_2d = jnp.reshape(eps2, (1, 1))
  x1, inv = _t1(eps1_2d, ndata, parts1, lin1_W, jnp.reshape(lin1_b, (1, D)))

  parts2 = _edge_kernel_128(x1, src, dst3, edge_w, zeros_128)
  parts2 = parts2.reshape(NC, N_NODES, D)
  x2 = _t2(eps2_2d, x1, parts2, inv, lin2_W, jnp.reshape(lin2_b, (1, D)))

  idx_pad = jnp.concatenate(
      [indices, jnp.zeros((B, IDXP - IDXW), jnp.int32)], axis=1)
  y = _sample_gather_kernel(x2, idx_pad.reshape(-1))

  flag = jnp.reshape(jnp.asarray(context, jnp.float32), (1, 1))
  out = _t3(flag, indices, y, c0_W, jnp.reshape(c0_b, (1, D)), c1_W,
            jnp.reshape(c1_b, (1, D // 2)), c2_W, jnp.reshape(c2_b, (1, 1)))
  return out


# speedup vs baseline: 6.9843x; 1.0008x over previous
import functools

import jax
import jax.numpy as jnp
from jax import lax
from jax.experimental import pallas as pl
from jax.experimental.pallas import tpu as pltpu
from jax.experimental.pallas import tpu_sc as plsc

N_NODES = 10000
N_EDGES = 320000
D = 128
B = 4096
IDXW = 23
IDXP = 24

NC = 2
NS = 16
NW = NC * NS
EDGES_PER_W = N_EDGES // NW
K = 80
NCHUNK = EDGES_PER_W // K
ROWS_PER_CP = 1000

SAMP_PER_W = B // NW
CH = 4
NCH = SAMP_PER_W // CH

_GDN = lax.GatherDimensionNumbers(
    offset_dims=(), collapsed_slice_dims=(0,), start_index_map=(0,))


def _bcast_lane(v, j):
  idx = jnp.full((16, 1), j, jnp.int32)
  return lax.gather(v, idx, _GDN, (1,),
                    mode=lax.GatherScatterMode.PROMISE_IN_BOUNDS)


def _make_edge_kernel(width):
  mesh = plsc.VectorSubcoreMesh(core_axis_name="c", subcore_axis_name="s")

  @functools.partial(
      pl.kernel,
      mesh=mesh,
      out_type=jax.ShapeDtypeStruct((NC * N_NODES, width), jnp.float32),
      compiler_params=pltpu.CompilerParams(use_tc_tiling_on_sc=False),
      scratch_types=[
          pltpu.VMEM((NCHUNK, K), jnp.int32),
          pltpu.VMEM((K,), jnp.int32),
          pltpu.VMEM((K,), jnp.int32),
          pltpu.VMEM((K,), jnp.float32),
          pltpu.VMEM((K,), jnp.float32),
          pltpu.VMEM((K, width), jnp.float32),
          pltpu.VMEM((K, width), jnp.float32),
          pltpu.VMEM_SHARED((N_NODES, width), jnp.float32),
          pltpu.SemaphoreType.DMA,
          pltpu.SemaphoreType.DMA,
          pltpu.SemaphoreType.DMA,
          pltpu.SemaphoreType.DMA,
          pltpu.SemaphoreType.DMA,
          pltpu.SemaphoreType.DMA,
      ],
  )
  def edge_kernel(table, src, dst3, w, zeros, out, dst_all, src0, src1, w0,
                  w1, rows0, rows1, agg_sh, gsem0, gsem1, ssem0, ssem1, wsem0,
                  wsem1):
    c = lax.axis_index("c")
    s = lax.axis_index("s")
    wid = c * NS + s
    @pl.when(s < N_NODES // ROWS_PER_CP)
    def _():
      pltpu.sync_copy(zeros.at[pl.ds(s * ROWS_PER_CP, ROWS_PER_CP)],
                      agg_sh.at[pl.ds(s * ROWS_PER_CP, ROWS_PER_CP)])
    base = wid * EDGES_PER_W
    pltpu.sync_copy(dst3.at[wid], dst_all)
    plsc.subcore_barrier()

    def src_start(i, buf, sem):
      pltpu.async_copy(src.at[pl.ds(base + i * K, K)], buf, sem)

    def src_wait(i, buf, sem):
      pltpu.make_async_copy(src.at[pl.ds(base + i * K, K)], buf, sem).wait()

    def w_start(i, buf, sem):
      pltpu.async_copy(w.at[pl.ds(base + i * K, K)], buf, sem)

    def w_wait(i, buf, sem):
      pltpu.make_async_copy(w.at[pl.ds(base + i * K, K)], buf, sem).wait()

    def g_start(sbuf, buf, sem):
      pltpu.async_copy(table.at[sbuf], buf, sem)

    def g_wait(sbuf, buf, sem):
      pltpu.make_async_copy(table.at[sbuf], buf, sem).wait()

    def mul_scatter(i, buf, wbuf):
      def group_body(g, carry2):
        wg = wbuf[pl.ds(g * 16, 16)]
        for j in range(16):
          wb = _bcast_lane(wg, j)
          e = g * 16 + j
          for t in range(D // 16):
            sl = buf[e, pl.ds(t * 16, 16)]
            buf[e, pl.ds(t * 16, 16)] = sl * wb
        return carry2

      lax.fori_loop(0, K // 16, group_body, 0)
      pltpu.sync_copy(buf, agg_sh.at[dst_all.at[i]], add=True)

    src_start(0, src0, ssem0)
    w_start(0, w0, wsem0)
    src_wait(0, src0, ssem0)
    g_start(src0, rows0, gsem0)
    src_start(1, src1, ssem1)
    w_start(1, w1, wsem1)

    def step(i, sbufs, wbufs, rbufs, ssems, wsems, gsems):
      src_wait(i + 1, sbufs[1], ssems[1])
      g_start(sbufs[1], rbufs[1], gsems[1])
      g_wait(sbufs[0], rbufs[0], gsems[0])

      @pl.when(i + 2 < NCHUNK)
      def _():
        src_start(i + 2, sbufs[0], ssems[0])

      w_wait(i, wbufs[0], wsems[0])
      mul_scatter(i, rbufs[0], wbufs[0])

      @pl.when(i + 2 < NCHUNK)
      def _():
        w_start(i + 2, wbufs[0], wsems[0])

    def pair_body(i2, carry):
      i = i2 * 2
      step(i, (src0, src1), (w0, w1), (rows0, rows1), (ssem0, ssem1),
           (wsem0, wsem1), (gsem0, gsem1))
      step(i + 1, (src1, src0), (w1, w0), (rows1, rows0), (ssem1, ssem0),
           (wsem1, wsem0), (gsem1, gsem0))
      return carry

    lax.fori_loop(0, (NCHUNK - 1) // 2, pair_body, 0)
    g_wait(src0, rows0, gsem0)
    w_wait(NCHUNK - 1, w0, wsem0)
    mul_scatter(NCHUNK - 1, rows0, w0)
    plsc.subcore_barrier()

    @pl.when(s < N_NODES // ROWS_PER_CP)
    def _():
      pltpu.sync_copy(
          agg_sh.at[pl.ds(s * ROWS_PER_CP, ROWS_PER_CP)],
          out.at[pl.ds(c * N_NODES + s * ROWS_PER_CP, ROWS_PER_CP)])

  return edge_kernel


_edge_kernel_pad = _make_edge_kernel(D + 16)
_edge_kernel_128 = _make_edge_kernel(D)


def _make_sample_gather_kernel():
  mesh = plsc.VectorSubcoreMesh(core_axis_name="c", subcore_axis_name="s")
  nidx = SAMP_PER_W * IDXP
  chi = CH * IDXP
  nring = 4

  @functools.partial(
      pl.kernel,
      mesh=mesh,
      out_type=jax.ShapeDtypeStruct((B, 3 * D), jnp.float32),
      compiler_params=pltpu.CompilerParams(use_tc_tiling_on_sc=False),
      scratch_types=[
          pltpu.VMEM((nidx,), jnp.int32),
          [pltpu.VMEM((chi,), jnp.int32) for _ in range(nring)],
          [pltpu.VMEM((chi, D), jnp.float32) for _ in range(nring)],
          pltpu.VMEM((SAMP_PER_W, 3 * D), jnp.float32),
          [pltpu.SemaphoreType.DMA for _ in range(nring)],
      ],
  )
  def gather_kernel(x, idx_flat, out, idx_v, idxb, rows, y_v, sems):
    c = lax.axis_index("c")
    s = lax.axis_index("s")
    wid = c * NS + s
    pltpu.sync_copy(idx_flat.at[pl.ds(wid * nidx, nidx)], idx_v)

    def g_start(i, b, sem):
      for k in range(chi // 16):
        idxb[b][pl.ds(k * 16, 16)] = idx_v[pl.ds(i * chi + k * 16, 16)]
      pltpu.async_copy(x.at[idxb[b]], rows[b], sem)

    def g_wait(b, sem):
      pltpu.make_async_copy(x.at[idxb[b]], rows[b], sem).wait()

    def process(i, buf):
      for j in range(CH):
        r0 = j * IDXP
        srow = i * CH + j
        accs = tuple(jnp.zeros((16,), jnp.float32) for _ in range(D // 16))

        def ctx_body(t, accs):
          return tuple(accs[k] + buf[r0 + t, pl.ds(k * 16, 16)]
                       for k in range(D // 16))

        accs = lax.fori_loop(3, IDXW, ctx_body, accs)
        for k in range(D // 16):
          sl = pl.ds(k * 16, 16)
          y_v[srow, pl.ds(k * 16, 16)] = buf[r0, sl]
          y_v[srow, pl.ds(D + k * 16, 16)] = buf[r0 + 1, sl]
          y_v[srow, pl.ds(2 * D + k * 16, 16)] = accs[k]

    for b in range(nring):
      g_start(b, b, sems[b])

    def ring_body(i4, carry):
      for b in range(nring):
        i = i4 * nring + b
        g_wait(b, sems[b])
        process(i, rows[b])

        @pl.when(i + nring < NCH)
        def _():
          g_start(i + nring, b, sems[b])
      return carry

    lax.fori_loop(0, NCH // nring, ring_body, 0)
    pltpu.sync_copy(y_v, out.at[pl.ds(wid * SAMP_PER_W, SAMP_PER_W)])

  return gather_kernel


_sample_gather_kernel = _make_sample_gather_kernel()


_BR = 1000


def _t1_body(eps_ref, x_ref, parts_ref, w_ref, b_ref, x1_ref, inv_ref):
  e = eps_ref[0, 0]
  p = parts_ref[0] + parts_ref[1]
  deg = p[:, D:D + 1]
  inv = 1.0 / jnp.where(deg == 0.0, 1.0, deg)
  h = x_ref[...] * (1.0 + e) + p[:, :D] * inv
  y = jnp.dot(h, w_ref[...], preferred_element_type=jnp.float32) + b_ref[...]
  x1_ref[...] = jnp.maximum(y, 0.0)
  inv_ref[...] = jnp.broadcast_to(inv, (_BR, D))


def _t1(eps1, ndata, parts, w1, b1):
  return pl.pallas_call(
      _t1_body,
      grid=(N_NODES // _BR,),
      in_specs=[
          pl.BlockSpec(memory_space=pltpu.SMEM),
          pl.BlockSpec((_BR, D), lambda i: (i, 0)),
          pl.BlockSpec((2, _BR, D + 16), lambda i: (0, i, 0)),
          pl.BlockSpec((D, D), lambda i: (0, 0)),
          pl.BlockSpec((1, D), lambda i: (0, 0)),
      ],
      out_specs=[
          pl.BlockSpec((_BR, D), lambda i: (i, 0)),
          pl.BlockSpec((_BR, D), lambda i: (i, 0)),
      ],
      out_shape=[
          jax.ShapeDtypeStruct((N_NODES, D), jnp.float32),
          jax.ShapeDtypeStruct((N_NODES, D), jnp.float32),
      ],
  )(eps1, ndata, parts, w1, b1)


def _t2_body(eps_ref, x_ref, parts_ref, inv_ref, w_ref, b_ref, x2_ref):
  e = eps_ref[0, 0]
  p = parts_ref[0] + parts_ref[1]
  h = x_ref[...] * (1.0 + e) + p * inv_ref[...]
  y = jnp.dot(h, w_ref[...], preferred_element_type=jnp.float32) + b_ref[...]

  @pl.when(pl.program_id(0) == 0)
  def _():
    y0 = jnp.where(
        lax.broadcasted_iota(jnp.int32, (_BR, D), 0) == 0, 0.0, y)
    x2_ref[...] = y0

  @pl.when(pl.program_id(0) != 0)
  def _():
    x2_ref[...] = y


def _t2(eps2, x1, parts2, inv, w2, b2):
  return pl.pallas_call(
      _t2_body,
      grid=(N_NODES // _BR,),
      in_specs=[
          pl.BlockSpec(memory_space=pltpu.SMEM),
          pl.BlockSpec((_BR, D), lambda i: (i, 0)),
          pl.BlockSpec((2, _BR, D), lambda i: (0, i, 0)),
          pl.BlockSpec((_BR, D), lambda i: (i, 0)),
          pl.BlockSpec((D, D), lambda i: (0, 0)),
          pl.BlockSpec((1, D), lambda i: (0, 0)),
      ],
      out_specs=pl.BlockSpec((_BR, D), lambda i: (i, 0)),
      out_shape=jax.ShapeDtypeStruct((N_NODES, D), jnp.float32),
  )(eps2, x1, parts2, inv, w2, b2)


def _t3_body(flag_ref, idx_ref, y_ref, c0w_ref, c0b_ref, c1w_ref, c1b_ref,
             c2w_ref, c2b_ref, out_ref):
  flag = flag_ref[0, 0]
  mask = (idx_ref[...][:, 3:] > 0).astype(jnp.float32)
  norm = jnp.sum(mask, axis=1, keepdims=True)
  inv = 1.0 / jnp.where(norm == 0.0, 1.0, norm)
  y = y_ref[...]
  ctx = y[:, 2 * D:] * (inv * flag)
  yy = jnp.concatenate([y[:, :2 * D], ctx], axis=1)
  h = jnp.maximum(
      jnp.dot(yy, c0w_ref[...], preferred_element_type=jnp.float32)
      + c0b_ref[...], 0.0)
  h = jnp.maximum(
      jnp.dot(h, c1w_ref[...], preferred_element_type=jnp.float32)
      + c1b_ref[...], 0.0)
  out_ref[...] = (
      jnp.dot(h, c2w_ref[...], preferred_element_type=jnp.float32)
      + c2b_ref[...])


def _t3(flag, indices, y, c0w, c0b, c1w, c1b, c2w, c2b):
  return pl.pallas_call(
      _t3_body,
      in_specs=[
          pl.BlockSpec(memory_space=pltpu.SMEM),
          pl.BlockSpec((B, IDXW), lambda: (0, 0)),
          pl.BlockSpec((B, 3 * D), lambda: (0, 0)),
          pl.BlockSpec((3 * D, D), lambda: (0, 0)),
          pl.BlockSpec((1, D), lambda: (0, 0)),
          pl.BlockSpec((D, D // 2), lambda: (0, 0)),
          pl.BlockSpec((1, D // 2), lambda: (0, 0)),
          pl.BlockSpec((D // 2, 1), lambda: (0, 0)),
          pl.BlockSpec((1, 1), lambda: (0, 0)),
      ],
      out_specs=pl.BlockSpec((B, 1), lambda: (0, 0)),
      out_shape=jax.ShapeDtypeStruct((B, 1), jnp.float32),
  )(flag, indices, y, c0w, c0b, c1w, c1b, c2w, c2b)


def kernel(ndata, edge_index, edge_w, eps1, eps2, lin1_W, lin1_b, lin2_W,
           lin2_b, c0_W, c0_b, c1_W, c1_b, c2_W, c2_b, indices, context):
  src = edge_index[0]
  dst3 = edge_index[1].reshape(NW, NCHUNK, K)
  xpad = jnp.concatenate(
      [ndata, jnp.ones((N_NODES, 1), jnp.float32),
       jnp.zeros((N_NODES, 15), jnp.float32)], axis=1)
  zeros_pad = jnp.zeros((N_NODES, D + 16), jnp.float32)
  zeros_128 = jnp.zeros((N_NODES, D), jnp.float32)

  parts1 = _edge_kernel_pad(xpad, src, dst3, edge_w, zeros_pad)
  parts1 = parts1.reshape(NC, N_NODES, D + 16)

  eps1_2d = jnp.reshape(eps1, (1, 1))
  eps2_2d = jnp.reshape(eps2, (1, 1))
  x1, inv = _t1(eps1_2d, ndata, parts1, lin1_W, jnp.reshape(lin1_b, (1, D)))

  parts2 = _edge_kernel_128(x1, src, dst3, edge_w, zeros_128)
  parts2 = parts2.reshape(NC, N_NODES, D)
  x2 = _t2(eps2_2d, x1, parts2, inv, lin2_W, jnp.reshape(lin2_b, (1, D)))

  idx_pad = jnp.concatenate(
      [indices, jnp.zeros((B, IDXP - IDXW), jnp.int32)], axis=1)
  y = _sample_gather_kernel(x2, idx_pad.reshape(-1))

  flag = jnp.reshape(jnp.asarray(context, jnp.float32), (1, 1))
  out = _t3(flag, indices, y, c0_W, jnp.reshape(c0_b, (1, D)), c1_W,
            jnp.reshape(c1_b, (1, D // 2)), c2_W, jnp.reshape(c2_b, (1, 1)))
  return out
